# baseline (speedup 1.0000x reference)
import functools

import jax
import jax.numpy as jnp
from jax import lax
from jax.experimental import pallas as pl
from jax.experimental.pallas import tpu as pltpu
from jax.experimental.pallas import tpu_sc as plsc

_LANES = 128
_NC = 2
_NS = 16


@functools.lru_cache(maxsize=None)
def _make_gather(n_rows: int, d: int, gpc: int):
    nw = _NC * _NS
    groups = n_rows // _LANES
    gpw = groups // nw
    n_chunks = gpw // gpc

    mesh = plsc.VectorSubcoreMesh(core_axis_name="c", subcore_axis_name="s")

    @functools.partial(
        pl.kernel,
        mesh=mesh,
        out_type=jax.ShapeDtypeStruct((n_rows, d), jnp.float32),
        scratch_types=[
            pltpu.VMEM((gpw, _LANES), jnp.int32),
            pltpu.VMEM((gpc * _LANES, d), jnp.float32),
            pltpu.SemaphoreType.DMA,
        ],
        compiler_params=pltpu.CompilerParams(use_tc_tiling_on_sc=False),
    )
    def gather_kernel(table_hbm, idx_hbm, out_hbm, idx_v, rows_v, sem):
        wid = lax.axis_index("s") * _NC + lax.axis_index("c")
        gbase = wid * gpw
        pltpu.sync_copy(idx_hbm.at[wid], idx_v)

        def chunk_body(ci, carry):
            copies = []
            for g in range(gpc):
                copies.append(
                    pltpu.async_copy(
                        table_hbm.at[idx_v.at[ci * gpc + g]],
                        rows_v.at[pl.ds(g * _LANES, _LANES)],
                        sem,
                    )
                )
            for c in copies:
                c.wait()
            pltpu.sync_copy(
                rows_v,
                out_hbm.at[pl.ds((gbase + ci * gpc) * _LANES, gpc * _LANES)],
            )
            return carry

        lax.fori_loop(0, n_chunks, chunk_body, 0)

    return gather_kernel


def kernel(table, input_ids):
    b, h = input_ids.shape
    d = table.shape[1]
    n = b * h
    nw = _NC * _NS
    idx = input_ids.reshape(nw, n // (_LANES * nw), _LANES).astype(jnp.int32)
    out = _make_gather(n, d, 10)(table, idx)
    return out.reshape(b, h, d)
